# baseline (speedup 1.0000x reference)
import functools

import jax
import jax.numpy as jnp
from jax import lax
from jax.experimental import pallas as pl
from jax.experimental.pallas import tpu as pltpu
from jax.experimental.pallas import tpu_sc as plsc

N_SC = 2
N_TILE = 16
LANES = 16
B = 96
NB = 9
CHUNK = NB * B

_f32 = jnp.float32
_i32 = jnp.int32


def _transform_matmul(x_pad, w0, w1, rel_coef, npad, d_in, d_out, n_rel):
    def wbody(coef_ref, w0_ref, w1_ref, out_ref):
        r = pl.program_id(0)
        out_ref[0] = (w0_ref[...] * coef_ref[r, 0]
                      + w1_ref[...] * coef_ref[r, 1])

    w_all = pl.pallas_call(
        wbody,
        grid=(n_rel,),
        in_specs=[
            pl.BlockSpec(memory_space=pltpu.SMEM),
            pl.BlockSpec((d_in, d_out), lambda r: (0, 0)),
            pl.BlockSpec((d_in, d_out), lambda r: (0, 0)),
        ],
        out_specs=pl.BlockSpec((1, d_in, d_out), lambda r: (r, 0, 0)),
        out_shape=jax.ShapeDtypeStruct((n_rel, d_in, d_out), _f32),
    )(rel_coef, w0, w1)

    blk_n = 10240
    nt = npad // blk_n

    def body(x_ref, w_ref, out_ref):
        out_ref[0] = jnp.dot(x_ref[...], w_ref[0],
                             preferred_element_type=_f32)

    return pl.pallas_call(
        body,
        grid=(nt, n_rel),
        in_specs=[
            pl.BlockSpec((blk_n, d_in), lambda n, r: (n, 0)),
            pl.BlockSpec((1, d_in, d_out), lambda n, r: (r, 0, 0)),
        ],
        out_specs=pl.BlockSpec((1, blk_n, d_out), lambda n, r: (r, n, 0)),
        out_shape=jax.ShapeDtypeStruct((n_rel, npad, d_out), _f32),
    )(x_pad, w_all)


def _counts_kernel(dst_all, rel_all, epad, kpad, n_rel):
    kslice = kpad // N_TILE
    zchunk = 2048
    nc = epad // (N_SC * N_TILE) // CHUNK
    mesh = plsc.VectorSubcoreMesh(core_axis_name="c", subcore_axis_name="s",
                                  num_cores=N_SC, num_subcores=N_TILE)

    @functools.partial(
        pl.kernel,
        mesh=mesh,
        out_type=jax.ShapeDtypeStruct((N_SC * kpad,), _f32),
        scratch_types=[
            pltpu.VMEM_SHARED((kpad,), _f32),
            pltpu.VMEM((CHUNK,), _i32),
            pltpu.VMEM((CHUNK,), _i32),
            pltpu.VMEM((NB, B), _i32),
            pltpu.VMEM((B,), _f32),
            pltpu.VMEM((zchunk,), _f32),
            pltpu.SemaphoreType.DMA,
        ],
    )
    def k(dst_hbm, rel_hbm, out_hbm, counts, sa, sb, key2d, onesv, zbuf,
          csem):
        c = lax.axis_index("c")
        s = lax.axis_index("s")

        def zfill(i, carry):
            zbuf[pl.ds(i * LANES, LANES)] = jnp.zeros((LANES,), _f32)
            return carry

        lax.fori_loop(0, zchunk // LANES, zfill, 0)
        koff = s * kslice
        for j in range(kslice // zchunk):
            pltpu.sync_copy(zbuf, counts.at[pl.ds(koff + j * zchunk, zchunk)])
        krem = kslice % zchunk
        if krem:
            pltpu.sync_copy(zbuf.at[pl.ds(0, krem)],
                            counts.at[pl.ds(koff + kslice - krem, krem)])
        for g in range(B // LANES):
            onesv[pl.ds(g * LANES, LANES)] = jnp.ones((LANES,), _f32)
        plsc.subcore_barrier()

        base0 = (c * N_TILE + s) * nc * CHUNK

        def cbody(ch, carry):
            base = base0 + ch * CHUNK
            pltpu.sync_copy(dst_hbm.at[pl.ds(base, CHUNK)], sa)
            pltpu.sync_copy(rel_hbm.at[pl.ds(base, CHUNK)], sb)
            for j in range(NB):
                for g in range(B // LANES):
                    sl = pl.ds(j * B + g * LANES, LANES)
                    key2d[j, pl.ds(g * LANES, LANES)] = (
                        sa[sl] * n_rel + sb[sl])
            for j in range(NB):
                pltpu.async_copy(onesv, counts.at[key2d.at[j]], csem,
                                 add=True).wait()
            return carry

        lax.fori_loop(0, nc, cbody, 0)
        plsc.subcore_barrier()
        nfull = kslice // zchunk
        krem2 = kslice % zchunk
        sizes = [zchunk] * nfull + ([krem2] if krem2 else [])
        for j, sz in enumerate(sizes):
            pltpu.sync_copy(counts.at[pl.ds(koff + j * zchunk, sz)],
                            zbuf.at[pl.ds(0, sz)])
            pltpu.sync_copy(zbuf.at[pl.ds(0, sz)],
                            out_hbm.at[pl.ds(c * kpad + koff + j * zchunk,
                                             sz)])

    return k(dst_all, rel_all)


def _agg_kernel(t_flat, cnt_flat, src_all, dst_all, rel_all, epad, kpad,
                npad, n_seeds, d_out, n_rel):
    rows_acc = ((n_seeds + 1 + 8 * N_TILE - 1) // (8 * N_TILE)) * (8 * N_TILE)
    kslice = kpad // N_TILE
    zrows = rows_acc // N_TILE
    zchunk = 2048
    nc2 = epad // (N_SC * N_TILE) // CHUNK
    nseg = d_out // LANES
    mesh = plsc.VectorSubcoreMesh(core_axis_name="c", subcore_axis_name="s",
                                  num_cores=N_SC, num_subcores=N_TILE)

    @functools.partial(
        pl.kernel,
        mesh=mesh,
        out_type=jax.ShapeDtypeStruct((N_SC, rows_acc, d_out), _f32),
        scratch_types=[
            pltpu.VMEM_SHARED((kpad,), _f32),
            pltpu.VMEM_SHARED((rows_acc, d_out), _f32),
            pltpu.VMEM((CHUNK,), _i32),
            pltpu.VMEM((CHUNK,), _i32),
            pltpu.VMEM((CHUNK,), _i32),
            pltpu.VMEM((NB, B), _i32),
            pltpu.VMEM((CHUNK,), _i32),
            pltpu.VMEM((CHUNK,), _f32),
            pltpu.VMEM((B,), _f32),
            pltpu.VMEM((zchunk,), _f32),
            pltpu.VMEM((zchunk,), _f32),
            pltpu.VMEM((B, d_out), _f32),
            pltpu.VMEM((B, d_out), _f32),
            pltpu.SemaphoreType.DMA,
            pltpu.SemaphoreType.DMA,
            pltpu.SemaphoreType.DMA,
            pltpu.SemaphoreType.DMA,
            pltpu.SemaphoreType.DMA,
        ],
    )
    def k(t_hbm, cnt_hbm, src_hbm, dst_hbm, rel_hbm, out_hbm,
          counts, accum, sa, sb, sc_, key2d, gidx, cnt2, normv, zbuf, zbuf2,
          rows_a, rows_b, gsem_a, gsem_b, ssem_a, ssem_b, csem):
        c = lax.axis_index("c")
        s = lax.axis_index("s")

        koff = s * kslice
        nfull = kslice // zchunk
        krem = kslice % zchunk
        sizes = [zchunk] * nfull + ([krem] if krem else [])
        for j, sz in enumerate(sizes):
            pltpu.sync_copy(cnt_hbm.at[pl.ds(koff + j * zchunk, sz)],
                            zbuf.at[pl.ds(0, sz)])
            pltpu.sync_copy(cnt_hbm.at[pl.ds(kpad + koff + j * zchunk, sz)],
                            zbuf2.at[pl.ds(0, sz)])
            for q in range(sz // LANES):
                sl = pl.ds(q * LANES, LANES)
                zbuf[sl] = zbuf[sl] + zbuf2[sl]
            pltpu.sync_copy(zbuf.at[pl.ds(0, sz)],
                            counts.at[pl.ds(koff + j * zchunk, sz)])

        def zfill2(e, carry):
            for g in range(nseg):
                rows_a[e, pl.ds(g * LANES, LANES)] = jnp.zeros((LANES,), _f32)
            return carry

        lax.fori_loop(0, B, zfill2, 0)
        roff = s * zrows
        for j in range(zrows // B):
            pltpu.sync_copy(rows_a, accum.at[pl.ds(roff + j * B, B), :])
        rrem = zrows % B
        if rrem:
            pltpu.sync_copy(rows_a.at[pl.ds(0, rrem), :],
                            accum.at[pl.ds(roff + zrows - rrem, rrem), :])
        plsc.subcore_barrier()

        base2 = (c * N_TILE + s) * nc2 * CHUNK

        def abody(ch, carry):
            base = base2 + ch * CHUNK
            d1 = pltpu.async_copy(src_hbm.at[pl.ds(base, CHUNK)], sa, csem)
            d2 = pltpu.async_copy(dst_hbm.at[pl.ds(base, CHUNK)], sb, csem)
            d3 = pltpu.async_copy(rel_hbm.at[pl.ds(base, CHUNK)], sc_, csem)
            d1.wait()
            d2.wait()
            d3.wait()
            for q in range(CHUNK // LANES):
                sl = pl.ds(q * LANES, LANES)
                gidx[sl] = sc_[sl] * npad + sa[sl]

            bufs = (rows_a, rows_b)
            gsems = (gsem_a, gsem_b)
            ssems = (ssem_a, ssem_b)
            gdesc = [None] * NB
            sdesc = [None] * NB
            gdesc[0] = pltpu.async_copy(t_hbm.at[gidx.at[pl.ds(0, B)]],
                                        bufs[0], gsems[0])
            for q in range(CHUNK // LANES):
                sl = pl.ds(q * LANES, LANES)
                sc_[sl] = sb[sl] * n_rel + sc_[sl]
            for j in range(NB):
                for g in range(B // LANES):
                    key2d[j, pl.ds(g * LANES, LANES)] = (
                        sb[pl.ds(j * B + g * LANES, LANES)])
            cdescs = []
            for j in range(NB):
                cdescs.append(pltpu.async_copy(
                    counts.at[sc_.at[pl.ds(j * B, B)]],
                    cnt2.at[pl.ds(j * B, B)], csem))

            for j in range(NB):
                cur = j % 2
                cdescs[j].wait()
                gdesc[j].wait()
                if j + 1 < NB:
                    if sdesc[j - 1] is not None:
                        sdesc[j - 1].wait()
                        sdesc[j - 1] = None
                    gdesc[j + 1] = pltpu.async_copy(
                        t_hbm.at[gidx.at[pl.ds((j + 1) * B, B)]],
                        bufs[1 - cur], gsems[1 - cur])
                for g in range(B // LANES):
                    sl = pl.ds(g * LANES, LANES)
                    normv[sl] = 1.0 / cnt2[pl.ds(j * B + g * LANES, LANES)]

                rows = bufs[cur]

                for grp in range(B // LANES):
                    nvec = normv[pl.ds(grp * LANES, LANES)]

                    def scale(e16, carry2, nvec=nvec, e0=grp * LANES):
                        nv = nvec.at[jnp.full((LANES,), e16, _i32)].get(
                            mode="promise_in_bounds")
                        e = e0 + e16
                        for g in range(nseg):
                            sl = pl.ds(g * LANES, LANES)
                            rows[e, sl] = nv * rows[e, sl]
                        return carry2

                    lax.fori_loop(0, LANES, scale, 0, unroll=8)
                sdesc[j] = pltpu.async_copy(rows, accum.at[key2d.at[j]],
                                            ssems[cur], add=True)
            for d in sdesc:
                if d is not None:
                    d.wait()
            return carry

        lax.fori_loop(0, nc2, abody, 0)
        plsc.subcore_barrier()
        pltpu.sync_copy(accum.at[pl.ds(s * zrows, zrows), :],
                        out_hbm.at[c, pl.ds(s * zrows, zrows), :])

    return k(t_flat, cnt_flat, src_all, dst_all, rel_all)


def _sum_partials(partials, n_seeds, d_out):
    blk = n_seeds // 10

    def body(p_ref, o_ref):
        o_ref[...] = p_ref[0] + p_ref[1]

    return pl.pallas_call(
        body,
        grid=(10,),
        in_specs=[pl.BlockSpec((N_SC, blk, d_out), lambda i: (0, i, 0))],
        out_specs=pl.BlockSpec((blk, d_out), lambda i: (i, 0)),
        out_shape=jax.ShapeDtypeStruct((n_seeds, d_out), _f32),
    )(partials)


def kernel(x, nodes, edge_index, edge_rel, rel_coef, rel_w):
    n_nodes, d_in = x.shape
    n_seeds = nodes.shape[0]
    n_edges = edge_index.shape[1]
    n_rel = rel_coef.shape[0]
    d_out = rel_w.shape[1]

    npad = ((n_nodes + 1023) // 1024) * 1024
    e_tot = n_edges + n_seeds
    chunk = N_SC * N_TILE * CHUNK
    epad = ((e_tot + chunk - 1) // chunk) * chunk
    pad = epad - e_tot
    kq = N_TILE * LANES
    kpad = (((n_seeds + 1) * n_rel + kq - 1) // kq) * kq

    src_all = jnp.concatenate([
        edge_index[0].astype(_i32), nodes.astype(_i32),
        jnp.zeros((pad,), _i32)])
    dst_all = jnp.concatenate([
        edge_index[1].astype(_i32), jnp.arange(n_seeds, dtype=_i32),
        jnp.full((pad,), n_seeds, _i32)])
    rel_all = jnp.concatenate([
        edge_rel.astype(_i32), jnp.zeros((n_seeds,), _i32),
        jnp.zeros((pad,), _i32)])

    x_pad = jnp.pad(x.astype(_f32), ((0, npad - n_nodes), (0, 0)))
    w0 = rel_w[:, :, 0].astype(_f32)
    w1 = rel_w[:, :, 1].astype(_f32)

    t_all = _transform_matmul(x_pad, w0, w1, rel_coef.astype(_f32),
                              npad, d_in, d_out, n_rel)
    t_flat = t_all.reshape(n_rel * npad, d_out)

    cnt_flat = _counts_kernel(dst_all, rel_all, epad, kpad, n_rel)
    partials = _agg_kernel(t_flat, cnt_flat, src_all, dst_all, rel_all, epad,
                           kpad, npad, n_seeds, d_out, n_rel)
    return _sum_partials(partials, n_seeds, d_out)


# speedup vs baseline: 1.0220x; 1.0220x over previous
import functools

import jax
import jax.numpy as jnp
from jax import lax
from jax.experimental import pallas as pl
from jax.experimental.pallas import tpu as pltpu
from jax.experimental.pallas import tpu_sc as plsc

N_SC = 2
N_TILE = 16
LANES = 16
B = 96
NB = 9
CHUNK = NB * B

_f32 = jnp.float32
_i32 = jnp.int32


def _transform_matmul(x_pad, w0, w1, rel_coef, npad, d_in, d_out, n_rel):
    def wbody(coef_ref, w0_ref, w1_ref, out_ref):
        r = pl.program_id(0)
        out_ref[0] = (w0_ref[...] * coef_ref[r, 0]
                      + w1_ref[...] * coef_ref[r, 1])

    w_all = pl.pallas_call(
        wbody,
        grid=(n_rel,),
        in_specs=[
            pl.BlockSpec(memory_space=pltpu.SMEM),
            pl.BlockSpec((d_in, d_out), lambda r: (0, 0)),
            pl.BlockSpec((d_in, d_out), lambda r: (0, 0)),
        ],
        out_specs=pl.BlockSpec((1, d_in, d_out), lambda r: (r, 0, 0)),
        out_shape=jax.ShapeDtypeStruct((n_rel, d_in, d_out), _f32),
    )(rel_coef, w0, w1)

    blk_n = 10240
    nt = npad // blk_n

    def body(x_ref, w_ref, out_ref):
        out_ref[0] = jnp.dot(x_ref[...], w_ref[0],
                             preferred_element_type=_f32)

    return pl.pallas_call(
        body,
        grid=(nt, n_rel),
        in_specs=[
            pl.BlockSpec((blk_n, d_in), lambda n, r: (n, 0)),
            pl.BlockSpec((1, d_in, d_out), lambda n, r: (r, 0, 0)),
        ],
        out_specs=pl.BlockSpec((1, blk_n, d_out), lambda n, r: (r, n, 0)),
        out_shape=jax.ShapeDtypeStruct((n_rel, npad, d_out), _f32),
    )(x_pad, w_all)


def _counts_kernel(dst_all, rel_all, epad, kpad, n_rel):
    kslice = kpad // N_TILE
    zchunk = 2048
    nc = epad // (N_SC * N_TILE) // CHUNK
    mesh = plsc.VectorSubcoreMesh(core_axis_name="c", subcore_axis_name="s",
                                  num_cores=N_SC, num_subcores=N_TILE)

    @functools.partial(
        pl.kernel,
        mesh=mesh,
        out_type=jax.ShapeDtypeStruct((N_SC * kpad,), _f32),
        scratch_types=[
            pltpu.VMEM_SHARED((kpad,), _f32),
            pltpu.VMEM((CHUNK,), _i32),
            pltpu.VMEM((CHUNK,), _i32),
            pltpu.VMEM((NB, B), _i32),
            pltpu.VMEM((B,), _f32),
            pltpu.VMEM((zchunk,), _f32),
            pltpu.SemaphoreType.DMA,
        ],
    )
    def k(dst_hbm, rel_hbm, out_hbm, counts, sa, sb, key2d, onesv, zbuf,
          csem):
        c = lax.axis_index("c")
        s = lax.axis_index("s")

        def zfill(i, carry):
            zbuf[pl.ds(i * LANES, LANES)] = jnp.zeros((LANES,), _f32)
            return carry

        lax.fori_loop(0, zchunk // LANES, zfill, 0)
        koff = s * kslice
        for j in range(kslice // zchunk):
            pltpu.sync_copy(zbuf, counts.at[pl.ds(koff + j * zchunk, zchunk)])
        krem = kslice % zchunk
        if krem:
            pltpu.sync_copy(zbuf.at[pl.ds(0, krem)],
                            counts.at[pl.ds(koff + kslice - krem, krem)])
        for g in range(B // LANES):
            onesv[pl.ds(g * LANES, LANES)] = jnp.ones((LANES,), _f32)
        plsc.subcore_barrier()

        base0 = (c * N_TILE + s) * nc * CHUNK

        def cbody(ch, carry):
            base = base0 + ch * CHUNK
            pltpu.sync_copy(dst_hbm.at[pl.ds(base, CHUNK)], sa)
            pltpu.sync_copy(rel_hbm.at[pl.ds(base, CHUNK)], sb)
            for j in range(NB):
                for g in range(B // LANES):
                    sl = pl.ds(j * B + g * LANES, LANES)
                    key2d[j, pl.ds(g * LANES, LANES)] = (
                        sa[sl] * n_rel + sb[sl])
            for j in range(NB):
                pltpu.async_copy(onesv, counts.at[key2d.at[j]], csem,
                                 add=True).wait()
            return carry

        lax.fori_loop(0, nc, cbody, 0)
        plsc.subcore_barrier()
        nfull = kslice // zchunk
        krem2 = kslice % zchunk
        sizes = [zchunk] * nfull + ([krem2] if krem2 else [])
        for j, sz in enumerate(sizes):
            pltpu.sync_copy(counts.at[pl.ds(koff + j * zchunk, sz)],
                            zbuf.at[pl.ds(0, sz)])
            pltpu.sync_copy(zbuf.at[pl.ds(0, sz)],
                            out_hbm.at[pl.ds(c * kpad + koff + j * zchunk,
                                             sz)])

    return k(dst_all, rel_all)


def _agg_kernel(t_flat, cnt_flat, src_all, dst_all, rel_all, epad, kpad,
                npad, n_seeds, d_out, n_rel):
    rows_acc = ((n_seeds + 1 + 8 * N_TILE - 1) // (8 * N_TILE)) * (8 * N_TILE)
    kslice = kpad // N_TILE
    zrows = rows_acc // N_TILE
    zchunk = 2048
    nc2 = epad // (N_SC * N_TILE) // CHUNK
    nseg = d_out // LANES
    mesh = plsc.VectorSubcoreMesh(core_axis_name="c", subcore_axis_name="s",
                                  num_cores=N_SC, num_subcores=N_TILE)

    @functools.partial(
        pl.kernel,
        mesh=mesh,
        out_type=jax.ShapeDtypeStruct((N_SC, rows_acc, d_out), _f32),
        scratch_types=[
            pltpu.VMEM_SHARED((kpad,), _f32),
            pltpu.VMEM_SHARED((rows_acc, d_out), _f32),
            pltpu.VMEM((CHUNK,), _i32),
            pltpu.VMEM((CHUNK,), _i32),
            pltpu.VMEM((CHUNK,), _i32),
            pltpu.VMEM((NB, B), _i32),
            pltpu.VMEM((CHUNK,), _i32),
            pltpu.VMEM((CHUNK,), _f32),
            pltpu.VMEM((B,), _f32),
            pltpu.VMEM((zchunk,), _f32),
            pltpu.VMEM((zchunk,), _f32),
            pltpu.VMEM((B, d_out), _f32),
            pltpu.VMEM((B, d_out), _f32),
            pltpu.SemaphoreType.DMA,
            pltpu.SemaphoreType.DMA,
            pltpu.SemaphoreType.DMA,
            pltpu.SemaphoreType.DMA,
            pltpu.SemaphoreType.DMA,
        ],
    )
    def k(t_hbm, cnt_hbm, src_hbm, dst_hbm, rel_hbm, out_hbm,
          counts, accum, sa, sb, sc_, key2d, gidx, cnt2, normv, zbuf, zbuf2,
          rows_a, rows_b, gsem_a, gsem_b, ssem_a, ssem_b, csem):
        c = lax.axis_index("c")
        s = lax.axis_index("s")

        koff = s * kslice
        nfull = kslice // zchunk
        krem = kslice % zchunk
        sizes = [zchunk] * nfull + ([krem] if krem else [])
        for j, sz in enumerate(sizes):
            pltpu.sync_copy(cnt_hbm.at[pl.ds(koff + j * zchunk, sz)],
                            zbuf.at[pl.ds(0, sz)])
            pltpu.sync_copy(cnt_hbm.at[pl.ds(kpad + koff + j * zchunk, sz)],
                            zbuf2.at[pl.ds(0, sz)])
            for q in range(sz // LANES):
                sl = pl.ds(q * LANES, LANES)
                zbuf[sl] = zbuf[sl] + zbuf2[sl]
            pltpu.sync_copy(zbuf.at[pl.ds(0, sz)],
                            counts.at[pl.ds(koff + j * zchunk, sz)])

        def zfill2(e, carry):
            for g in range(nseg):
                rows_a[e, pl.ds(g * LANES, LANES)] = jnp.zeros((LANES,), _f32)
            return carry

        lax.fori_loop(0, B, zfill2, 0)
        roff = s * zrows
        for j in range(zrows // B):
            pltpu.sync_copy(rows_a, accum.at[pl.ds(roff + j * B, B), :])
        rrem = zrows % B
        if rrem:
            pltpu.sync_copy(rows_a.at[pl.ds(0, rrem), :],
                            accum.at[pl.ds(roff + zrows - rrem, rrem), :])
        plsc.subcore_barrier()

        base2 = (c * N_TILE + s) * nc2 * CHUNK

        def abody(ch, carry):
            base = base2 + ch * CHUNK
            d1 = pltpu.async_copy(src_hbm.at[pl.ds(base, CHUNK)], sa, csem)
            d2 = pltpu.async_copy(dst_hbm.at[pl.ds(base, CHUNK)], sb, csem)
            d3 = pltpu.async_copy(rel_hbm.at[pl.ds(base, CHUNK)], sc_, csem)
            d1.wait()
            d2.wait()
            d3.wait()
            for q in range(CHUNK // LANES):
                sl = pl.ds(q * LANES, LANES)
                gidx[sl] = sc_[sl] * npad + sa[sl]

            bufs = (rows_a, rows_b)
            gsems = (gsem_a, gsem_b)
            ssems = (ssem_a, ssem_b)
            gdesc = [None] * NB
            sdesc = [None] * NB
            gdesc[0] = pltpu.async_copy(t_hbm.at[gidx.at[pl.ds(0, B)]],
                                        bufs[0], gsems[0])
            for q in range(CHUNK // LANES):
                sl = pl.ds(q * LANES, LANES)
                sc_[sl] = sb[sl] * n_rel + sc_[sl]
            for j in range(NB):
                for g in range(B // LANES):
                    key2d[j, pl.ds(g * LANES, LANES)] = (
                        sb[pl.ds(j * B + g * LANES, LANES)])
            cdescs = []
            for j in range(NB):
                cdescs.append(pltpu.async_copy(
                    counts.at[sc_.at[pl.ds(j * B, B)]],
                    cnt2.at[pl.ds(j * B, B)], csem))

            for j in range(NB):
                cur = j % 2
                cdescs[j].wait()
                gdesc[j].wait()
                if j + 1 < NB:
                    if sdesc[j - 1] is not None:
                        sdesc[j - 1].wait()
                        sdesc[j - 1] = None
                    gdesc[j + 1] = pltpu.async_copy(
                        t_hbm.at[gidx.at[pl.ds((j + 1) * B, B)]],
                        bufs[1 - cur], gsems[1 - cur])
                for g in range(B // LANES):
                    sl = pl.ds(g * LANES, LANES)
                    normv[sl] = 1.0 / cnt2[pl.ds(j * B + g * LANES, LANES)]

                rows = bufs[cur]

                for grp in range(B // LANES):
                    nvec = normv[pl.ds(grp * LANES, LANES)]

                    def scale(e16, carry2, nvec=nvec, e0=grp * LANES):
                        nv = nvec.at[jnp.full((LANES,), e16, _i32)].get(
                            mode="promise_in_bounds")
                        e = e0 + e16
                        for g in range(nseg):
                            sl = pl.ds(g * LANES, LANES)
                            rows[e, sl] = nv * rows[e, sl]
                        return carry2

                    lax.fori_loop(0, LANES, scale, 0, unroll=4)
                sdesc[j] = pltpu.async_copy(rows, accum.at[key2d.at[j]],
                                            ssems[cur], add=True)
            for d in sdesc:
                if d is not None:
                    d.wait()
            return carry

        lax.fori_loop(0, nc2, abody, 0)
        plsc.subcore_barrier()
        pltpu.sync_copy(accum.at[pl.ds(s * zrows, zrows), :],
                        out_hbm.at[c, pl.ds(s * zrows, zrows), :])

    return k(t_flat, cnt_flat, src_all, dst_all, rel_all)


def _sum_partials(partials, n_seeds, d_out):
    blk = n_seeds // 10

    def body(p_ref, o_ref):
        o_ref[...] = p_ref[0] + p_ref[1]

    return pl.pallas_call(
        body,
        grid=(10,),
        in_specs=[pl.BlockSpec((N_SC, blk, d_out), lambda i: (0, i, 0))],
        out_specs=pl.BlockSpec((blk, d_out), lambda i: (i, 0)),
        out_shape=jax.ShapeDtypeStruct((n_seeds, d_out), _f32),
    )(partials)


def kernel(x, nodes, edge_index, edge_rel, rel_coef, rel_w):
    n_nodes, d_in = x.shape
    n_seeds = nodes.shape[0]
    n_edges = edge_index.shape[1]
    n_rel = rel_coef.shape[0]
    d_out = rel_w.shape[1]

    npad = ((n_nodes + 1023) // 1024) * 1024
    e_tot = n_edges + n_seeds
    chunk = N_SC * N_TILE * CHUNK
    epad = ((e_tot + chunk - 1) // chunk) * chunk
    pad = epad - e_tot
    kq = N_TILE * LANES
    kpad = (((n_seeds + 1) * n_rel + kq - 1) // kq) * kq

    src_all = jnp.concatenate([
        edge_index[0].astype(_i32), nodes.astype(_i32),
        jnp.zeros((pad,), _i32)])
    dst_all = jnp.concatenate([
        edge_index[1].astype(_i32), jnp.arange(n_seeds, dtype=_i32),
        jnp.full((pad,), n_seeds, _i32)])
    rel_all = jnp.concatenate([
        edge_rel.astype(_i32), jnp.zeros((n_seeds,), _i32),
        jnp.zeros((pad,), _i32)])

    x_pad = jnp.pad(x.astype(_f32), ((0, npad - n_nodes), (0, 0)))
    w0 = rel_w[:, :, 0].astype(_f32)
    w1 = rel_w[:, :, 1].astype(_f32)

    t_all = _transform_matmul(x_pad, w0, w1, rel_coef.astype(_f32),
                              npad, d_in, d_out, n_rel)
    t_flat = t_all.reshape(n_rel * npad, d_out)

    cnt_flat = _counts_kernel(dst_all, rel_all, epad, kpad, n_rel)
    partials = _agg_kernel(t_flat, cnt_flat, src_all, dst_all, rel_all, epad,
                           kpad, npad, n_seeds, d_out, n_rel)
    return _sum_partials(partials, n_seeds, d_out)
